# Initial kernel scaffold; baseline (speedup 1.0000x reference)
import jax
import jax.numpy as jnp
from jax.experimental import pallas as pl


def kernel(xyz_normed, plane0, plane1, plane2, line0, line1, line2):
    raise NotImplementedError("write your pallas kernel here")


# speedup vs baseline: 31.9463x; 31.9463x over previous
import functools

import jax
import jax.numpy as jnp
from jax import lax
from jax.experimental import pallas as pl
from jax.experimental.pallas import tpu as pltpu
from jax.experimental.pallas import tpu_sc as plsc

GRID = 300
DIM = 48
N = 262144
L = 16
B = 128
NW = 32
PPW = N // NW
NCHUNK = PPW // B
MODES = ((0, 1, 2), (0, 2, 1), (1, 2, 0))


def _sc_body(xs, ys, zs, pt0, pt1, pt2, lt0, lt1, lt2,
             o0, o1, o2,
             xv, yv, zv,
             i00, i01, i10, i11,
             wa, wb, wc, wd,
             li0, li1, lw0, lw1,
             g00, g01, g10, g11,
             tl0, tl1, tl2, ot, sem):
    wid = lax.axis_index("s") * 2 + lax.axis_index("c")
    base = wid * PPW
    coords = (xv, yv, zv)
    planes = (pt0, pt1, pt2)
    ltabs = (tl0, tl1, tl2)
    outs = (o0, o1, o2)

    pltpu.sync_copy(lt0, tl0)
    pltpu.sync_copy(lt1, tl1)
    pltpu.sync_copy(lt2, tl2)

    def chunk_body(k, _):
        p0 = base + k * B
        pltpu.sync_copy(xs.at[pl.ds(p0, B)], xv)
        pltpu.sync_copy(ys.at[pl.ds(p0, B)], yv)
        pltpu.sync_copy(zs.at[pl.ds(p0, B)], zv)

        for m in range(3):
            ub, hb, lb = (coords[c] for c in MODES[m])

            def idx_body(g, _, ub=ub, hb=hb, lb=lb):
                s = g * L
                u = ub[pl.ds(s, L)]
                h = hb[pl.ds(s, L)]
                v = lb[pl.ds(s, L)]
                ix = (u + 1.0) * 0.5 * (GRID - 1)
                iy = (h + 1.0) * 0.5 * (GRID - 1)
                iv = (v + 1.0) * 0.5 * (GRID - 1)
                x0 = jnp.minimum(ix.astype(jnp.int32), GRID - 2)
                y0 = jnp.minimum(iy.astype(jnp.int32), GRID - 2)
                v0 = jnp.minimum(iv.astype(jnp.int32), GRID - 2)
                wx = ix - x0.astype(jnp.float32)
                wy = iy - y0.astype(jnp.float32)
                wv = iv - v0.astype(jnp.float32)
                r00 = y0 * GRID + x0
                i00[pl.ds(s, L)] = r00
                i01[pl.ds(s, L)] = r00 + 1
                i10[pl.ds(s, L)] = r00 + GRID
                i11[pl.ds(s, L)] = r00 + GRID + 1
                ex = 1.0 - wx
                ey = 1.0 - wy
                wa[pl.ds(s, L)] = ex * ey
                wb[pl.ds(s, L)] = wx * ey
                wc[pl.ds(s, L)] = ex * wy
                wd[pl.ds(s, L)] = wx * wy
                li0[pl.ds(s, L)] = v0
                li1[pl.ds(s, L)] = v0 + 1
                lw0[pl.ds(s, L)] = 1.0 - wv
                lw1[pl.ds(s, L)] = wv
                return 0

            lax.fori_loop(0, B // L, idx_body, 0, unroll=False)

            h0 = pltpu.async_copy(planes[m].at[i00], g00, sem)
            h1 = pltpu.async_copy(planes[m].at[i01], g01, sem)
            h2 = pltpu.async_copy(planes[m].at[i10], g10, sem)
            h3 = pltpu.async_copy(planes[m].at[i11], g11, sem)
            h0.wait()
            h1.wait()
            h2.wait()
            h3.wait()

            tl = ltabs[m]

            def pt_body(b, _, tl=tl):
                bs = jnp.full((L,), b, jnp.int32)
                w00 = plsc.load_gather(wa, [bs])
                w01 = plsc.load_gather(wb, [bs])
                w10 = plsc.load_gather(wc, [bs])
                w11 = plsc.load_gather(wd, [bs])
                l0w = plsc.load_gather(lw0, [bs])
                l1w = plsc.load_gather(lw1, [bs])
                r0 = plsc.load_gather(li0, [bs]) * DIM
                r1 = plsc.load_gather(li1, [bs]) * DIM
                ci = lax.iota(jnp.int32, L)
                for cg in range(DIM // L):
                    sl = pl.ds(cg * L, L)
                    v00 = g00[b, sl]
                    v01 = g01[b, sl]
                    v10 = g10[b, sl]
                    v11 = g11[b, sl]
                    t0 = plsc.load_gather(tl, [r0 + (ci + cg * L)])
                    t1 = plsc.load_gather(tl, [r1 + (ci + cg * L)])
                    pcv = v00 * w00 + v01 * w01 + v10 * w10 + v11 * w11
                    lcv = t0 * l0w + t1 * l1w
                    ot[b, sl] = pcv * lcv
                return 0

            lax.fori_loop(0, B, pt_body, 0, unroll=False)
            pltpu.sync_copy(ot, outs[m].at[pl.ds(p0, B)])
        return 0

    lax.fori_loop(0, NCHUNK, chunk_body, 0, unroll=False)


@functools.cache
def _build_sc_call():
  return functools.partial(
    pl.kernel,
    out_type=tuple(jax.ShapeDtypeStruct((N, DIM), jnp.float32) for _ in range(3)),
    mesh=plsc.VectorSubcoreMesh(core_axis_name="c", subcore_axis_name="s"),
    compiler_params=pltpu.CompilerParams(needs_layout_passes=False, use_tc_tiling_on_sc=False),
    scratch_types=[
        pltpu.VMEM((B,), jnp.float32),
        pltpu.VMEM((B,), jnp.float32),
        pltpu.VMEM((B,), jnp.float32),
        pltpu.VMEM((B,), jnp.int32),
        pltpu.VMEM((B,), jnp.int32),
        pltpu.VMEM((B,), jnp.int32),
        pltpu.VMEM((B,), jnp.int32),
        pltpu.VMEM((B,), jnp.float32),
        pltpu.VMEM((B,), jnp.float32),
        pltpu.VMEM((B,), jnp.float32),
        pltpu.VMEM((B,), jnp.float32),
        pltpu.VMEM((B,), jnp.int32),
        pltpu.VMEM((B,), jnp.int32),
        pltpu.VMEM((B,), jnp.float32),
        pltpu.VMEM((B,), jnp.float32),
        pltpu.VMEM((B, DIM), jnp.float32),
        pltpu.VMEM((B, DIM), jnp.float32),
        pltpu.VMEM((B, DIM), jnp.float32),
        pltpu.VMEM((B, DIM), jnp.float32),
        pltpu.VMEM((GRID * DIM,), jnp.float32),
        pltpu.VMEM((GRID * DIM,), jnp.float32),
        pltpu.VMEM((GRID * DIM,), jnp.float32),
        pltpu.VMEM((B, DIM), jnp.float32),
        pltpu.SemaphoreType.DMA,
    ],
  )(_sc_body)


def kernel(xyz_normed, plane0, plane1, plane2, line0, line1, line2):
    xs = xyz_normed[:, 0]
    ys = xyz_normed[:, 1]
    zs = xyz_normed[:, 2]
    pts = [jnp.transpose(p, (1, 2, 0)).reshape(GRID * GRID, DIM)
           for p in (plane0, plane1, plane2)]
    lts = [jnp.transpose(l[:, :, 0], (1, 0)).reshape(GRID * DIM) for l in (line0, line1, line2)]
    f0, f1, f2 = _build_sc_call()(xs, ys, zs, *pts, *lts)
    return (f0.T, f1.T, f2.T)
